# baseline (speedup 1.0000x reference)
import functools

import jax
import jax.numpy as jnp
from jax import lax
from jax.experimental import pallas as pl
from jax.experimental.pallas import tpu as pltpu
from jax.experimental.pallas import tpu_sc as plsc

VOCAB = 1000000
EMBD = 32
B = 4096
L = 200

NC = 2
NS = 16
NW = NC * NS

TOTAL = B * L
PER_W = TOTAL // NW
R = 128
NBLK = PER_W // R
K = 4
GROUP = K * R
NGRP = NBLK // K
OROW = GROUP * EMBD // 128
ONR = TOTAL * EMBD // 128


def _make_kernel():
    mesh = plsc.VectorSubcoreMesh(core_axis_name="c", subcore_axis_name="s")

    @functools.partial(
        pl.kernel,
        out_type=jax.ShapeDtypeStruct((ONR, 128), jnp.float32),
        mesh=mesh,
        compiler_params=pltpu.CompilerParams(use_tc_tiling_on_sc=False),
        scratch_types=[
            pltpu.VMEM((NBLK, R), jnp.int32),
            pltpu.VMEM((GROUP, EMBD), jnp.float32),
            pltpu.VMEM((GROUP, EMBD), jnp.float32),
            pltpu.VMEM((OROW, 128), jnp.float32),
            pltpu.VMEM((OROW, 128), jnp.float32),
            pltpu.SemaphoreType.DMA,
            pltpu.SemaphoreType.DMA,
            pltpu.SemaphoreType.DMA,
            pltpu.SemaphoreType.DMA,
        ],
    )
    def emb_kernel(table_hbm, x_hbm, out_hbm,
                   idx_v, gb0, gb1, ob0, ob1, g0, g1, s0, s1):
        gbuf = (gb0, gb1)
        obuf = (ob0, ob1)
        gsem = (g0, g1)
        ssem = (s0, s1)
        wid = lax.axis_index("s") * NC + lax.axis_index("c")
        obase = wid * (PER_W * EMBD // 128)
        pltpu.sync_copy(x_hbm.at[wid], idx_v)

        def gather_start(g, s):
            for i in range(K):
                pltpu.async_copy(
                    table_hbm.at[idx_v.at[g * K + i]],
                    gbuf[s].at[pl.ds(i * R, R)],
                    gsem[s],
                )

        def gather_wait(g, s):
            for i in range(K):
                pltpu.make_async_copy(
                    table_hbm.at[idx_v.at[g * K + i]],
                    gbuf[s].at[pl.ds(i * R, R)],
                    gsem[s],
                ).wait()

        def store_start(g, s):
            pltpu.async_copy(
                obuf[s], out_hbm.at[pl.ds(obase + g * OROW, OROW)], ssem[s]
            )

        def store_wait(g, s):
            pltpu.make_async_copy(
                obuf[s], out_hbm.at[pl.ds(obase + g * OROW, OROW)], ssem[s]
            ).wait()

        def relu_repack(s):
            src = gbuf[s]
            dst = obuf[s]

            @pl.loop(0, OROW, unroll=2)
            def _rows(q):
                for c in range(8):
                    dst[q, c * 16:c * 16 + 16] = jnp.maximum(
                        src[4 * q + c // 2, (c % 2) * 16:(c % 2) * 16 + 16],
                        0.0,
                    )

        gather_start(0, 0)

        @pl.loop(0, NGRP, step=2)
        def _pair(G):
            for s in range(2):
                g = G + s
                o = 1 - s

                @pl.when(g >= 1)
                def _drain_prev_store():
                    store_wait(g - 1, o)

                @pl.when(g + 1 < NGRP)
                def _fire_next_gather():
                    gather_start(g + 1, o)

                gather_wait(g, s)
                relu_repack(s)
                store_start(g, s)

        store_wait(NGRP - 1, 1)

    return emb_kernel


_EMB_KERNEL = _make_kernel()


@jax.jit
def kernel(x, table):
    x_flat = x.astype(jnp.int32).reshape(NW, NBLK, R)
    out = _EMB_KERNEL(table, x_flat)
    return out.reshape(B, L, EMBD)
